# baseline (speedup 1.0000x reference)
import functools

import jax
import jax.numpy as jnp
from jax import lax
from jax.experimental import pallas as pl
from jax.experimental.pallas import tpu as pltpu
from jax.experimental.pallas import tpu_sc as plsc


def _rsqrt_newton(sv):
    ih = lax.bitcast_convert_type(sv, jnp.int32)
    ih = jnp.int32(0x5F3759DF) - lax.shift_right_logical(ih, 1)
    y = lax.bitcast_convert_type(ih, jnp.float32)
    for _ in range(3):
        y = y * (1.5 - 0.5 * sv * y * y)
    return y


def kernel(task_ids, table):
    B, = task_ids.shape
    V, D = table.shape
    info = plsc.get_sparse_core_info()
    NC, NS, L = info.num_cores, info.num_subcores, info.num_lanes
    NW = NC * NS
    b_per_w = B // NW
    CHUNK = 128
    n_chunks = b_per_w // CHUNK
    n_half = D // L
    PACK = 128 // D
    shift = PACK.bit_length() - 1

    tab2 = table.reshape(V // PACK, PACK * D)
    mesh = plsc.VectorSubcoreMesh(core_axis_name="c", subcore_axis_name="s")

    @functools.partial(
        pl.kernel,
        out_type=jax.ShapeDtypeStruct((B, D), jnp.float32),
        mesh=mesh,
        compiler_params=pltpu.CompilerParams(needs_layout_passes=False),
        scratch_types=[
            pltpu.VMEM((n_chunks, CHUNK), jnp.int32),
            pltpu.VMEM((n_chunks, CHUNK), jnp.int32),
            pltpu.VMEM((2, CHUNK, PACK * D), jnp.float32),
            pltpu.VMEM((2, CHUNK, D), jnp.float32),
            pltpu.SemaphoreType.DMA,
            pltpu.SemaphoreType.DMA,
        ],
    )
    def sc_kernel(idx_hbm, table_hbm, out_hbm, idx_v, row_v, gbuf, obuf,
                  gsem, osem):
        wid = lax.axis_index("s") * NC + lax.axis_index("c")
        base = wid * b_per_w

        for c in range(n_chunks):
            pltpu.sync_copy(idx_hbm.at[pl.ds(base + c * CHUNK, CHUNK)],
                            idx_v.at[c])
            for j in range(CHUNK // L):
                ids = idx_v[c, pl.ds(j * L, L)]
                row_v[c, pl.ds(j * L, L)] = lax.shift_right_logical(
                    ids, shift)

        def start_gather(c):
            return pltpu.async_copy(
                table_hbm.at[row_v.at[c]], gbuf.at[c % 2], gsem)

        def normalize_chunk(c):
            slot = c % 2
            gb = gbuf.at[slot]
            ob = obuf.at[slot]

            def body(j, _):
                ids = idx_v[c, pl.ds(j * L, L)]
                cols = (ids & (PACK - 1)) * D
                for r in range(L):
                    col = cols[r]
                    i = j * L + r
                    halves = [gb[i, pl.ds(col + h * L, L)]
                              for h in range(n_half)]
                    sq = halves[0] * halves[0]
                    for h in range(1, n_half):
                        sq = sq + halves[h] * halves[h]
                    s = jnp.sum(sq)
                    sv = lax.broadcast_in_dim(s, (L,), ())
                    sv = jnp.maximum(sv, 1e-24)
                    y = _rsqrt_newton(sv)
                    for h in range(n_half):
                        ob[i, pl.ds(h * L, L)] = halves[h] * y
                return 0

            lax.fori_loop(0, CHUNK // L, body, 0)

        gathers = [start_gather(0)]
        out_copies = []
        for c in range(n_chunks):
            gathers[c].wait()
            if c >= 1:
                out_copies[c - 1].wait()
            if c + 1 < n_chunks:
                gathers.append(start_gather(c + 1))
            normalize_chunk(c)
            out_copies.append(pltpu.async_copy(
                obuf.at[c % 2],
                out_hbm.at[pl.ds(base + c * CHUNK, CHUNK)], osem))
        out_copies[-1].wait()

    return sc_kernel(task_ids, tab2)


# speedup vs baseline: 1.0305x; 1.0305x over previous
import functools

import jax
import jax.numpy as jnp
from jax import lax
from jax.experimental import pallas as pl
from jax.experimental.pallas import tpu as pltpu
from jax.experimental.pallas import tpu_sc as plsc


def _rsqrt_newton(sv):
    ih = lax.bitcast_convert_type(sv, jnp.int32)
    ih = jnp.int32(0x5F3759DF) - lax.shift_right_logical(ih, 1)
    y = lax.bitcast_convert_type(ih, jnp.float32)
    for _ in range(3):
        y = y * (1.5 - 0.5 * sv * y * y)
    return y


def kernel(task_ids, table):
    B, = task_ids.shape
    V, D = table.shape
    info = plsc.get_sparse_core_info()
    NC, NS, L = info.num_cores, info.num_subcores, info.num_lanes
    NW = NC * NS
    b_per_w = B // NW
    CHUNK = 128
    n_chunks = b_per_w // CHUNK
    PK = 128 // D

    tab_flat = jnp.reshape(table, (V // PK, PK * D))

    mesh = plsc.VectorSubcoreMesh(core_axis_name="c", subcore_axis_name="s")

    @functools.partial(
        pl.kernel,
        out_type=jax.ShapeDtypeStruct((D, B), jnp.float32),
        mesh=mesh,
        compiler_params=pltpu.CompilerParams(needs_layout_passes=False),
        scratch_types=[
            pltpu.VMEM((n_chunks, CHUNK), jnp.int32),
            pltpu.VMEM((n_chunks, CHUNK), jnp.int32),
            pltpu.VMEM((2, CHUNK, PK * D), jnp.float32),
            pltpu.VMEM((2, D, CHUNK), jnp.float32),
            pltpu.SemaphoreType.DMA,
            pltpu.SemaphoreType.DMA,
        ],
    )
    def sc_kernel(idx_hbm, tab_hbm, out_hbm, idx_v, row_v, gbuf, obuf,
                  gsem, osem):
        wid = lax.axis_index("s") * NC + lax.axis_index("c")
        base = wid * b_per_w

        for c in range(n_chunks):
            pltpu.sync_copy(idx_hbm.at[pl.ds(base + c * CHUNK, CHUNK)],
                            idx_v.at[c])

        def rows_body(i, _):
            c = i // (CHUNK // L)
            t = i % (CHUNK // L)
            ts = pl.ds(t * L, L)
            row_v[c, ts] = lax.shift_right_logical(idx_v[c, ts], 2)
            return 0

        lax.fori_loop(0, n_chunks * (CHUNK // L), rows_body, 0)

        def start_gather(c):
            return pltpu.async_copy(
                tab_hbm.at[row_v.at[c]], gbuf.at[c % 2], gsem)

        def process_chunk(c):
            slot = c % 2
            gb = gbuf.at[slot]
            ob = obuf.at[slot]
            riota = lax.iota(jnp.int32, L)

            def body(t, _):
                ts = pl.ds(t * L, L)
                rows = riota + t * L
                sub = (idx_v[c, ts] & (PK - 1)) * D
                feats = []
                sq = None
                for f in range(D):
                    x = plsc.load_gather(gb, [rows, sub + f])
                    feats.append(x)
                    sq = x * x if sq is None else sq + x * x
                y = _rsqrt_newton(jnp.maximum(sq, 1e-24))
                for f in range(D):
                    ob[f, ts] = feats[f] * y
                return 0

            lax.fori_loop(0, CHUNK // L, body, 0)

        out_copies = []
        gather = start_gather(0)
        for c in range(n_chunks):
            gather.wait()
            if c >= 1:
                out_copies[c - 1].wait()
            if c + 1 < n_chunks:
                gather = start_gather(c + 1)
            process_chunk(c)
            out_copies.append(pltpu.async_copy(
                obuf.at[c % 2],
                out_hbm.at[:, pl.ds(base + c * CHUNK, CHUNK)], osem))
        out_copies[-1].wait()

    return sc_kernel(task_ids.astype(jnp.int32), tab_flat).T


# speedup vs baseline: 2.2363x; 2.1701x over previous
import functools

import jax
import jax.numpy as jnp
from jax import lax
from jax.experimental import pallas as pl
from jax.experimental.pallas import tpu as pltpu
from jax.experimental.pallas import tpu_sc as plsc


def _rsqrt_newton(sv):
    ih = lax.bitcast_convert_type(sv, jnp.int32)
    ih = jnp.int32(0x5F3759DF) - lax.shift_right_logical(ih, 1)
    y = lax.bitcast_convert_type(ih, jnp.float32)
    for _ in range(3):
        y = y * (1.5 - 0.5 * sv * y * y)
    return y


def kernel(task_ids, table):
    B, = task_ids.shape
    V, D = table.shape
    info = plsc.get_sparse_core_info()
    NC, NS, L = info.num_cores, info.num_subcores, info.num_lanes
    NW = NC * NS
    b_per_w = B // NW
    CHUNK = 128
    n_chunks = b_per_w // CHUNK
    BLK = 128

    mesh = plsc.VectorSubcoreMesh(core_axis_name="c", subcore_axis_name="s")

    @functools.partial(
        pl.kernel,
        out_type=jax.ShapeDtypeStruct((D, B), jnp.float32),
        mesh=mesh,
        compiler_params=pltpu.CompilerParams(needs_layout_passes=False),
        scratch_types=[
            pltpu.VMEM((b_per_w,), jnp.int32),
            pltpu.VMEM((2, D, BLK), jnp.float32),
            pltpu.VMEM((2, D, CHUNK), jnp.float32),
            pltpu.SemaphoreType.DMA,
            pltpu.SemaphoreType.DMA,
            pltpu.SemaphoreType.DMA,
        ],
    )
    def sc_kernel(idx_hbm, tab_hbm, out_hbm, idx_v, gbuf, obuf,
                  gsem0, gsem1, osem):
        wid = lax.axis_index("s") * NC + lax.axis_index("c")
        base = wid * b_per_w
        gsems = (gsem0, gsem1)

        pltpu.sync_copy(idx_hbm.at[pl.ds(base, b_per_w)], idx_v)
        liota = lax.iota(jnp.int32, L)

        def read_idx(jj):
            grp = idx_v[pl.ds((jj // L) * L, L)]
            return jnp.sum(jnp.where(liota == jj % L, grp, 0))

        def fetch(jj, slot):
            v = read_idx(jnp.minimum(jj, b_per_w - 1))
            vb = pl.multiple_of((v >> 7) * BLK, BLK)
            pltpu.async_copy(tab_hbm.at[:, pl.ds(vb, BLK)],
                             gbuf.at[slot], gsems[slot])

        def drain(slot):
            pltpu.make_async_copy(tab_hbm.at[:, pl.ds(0, BLK)],
                                  gbuf.at[slot], gsems[slot]).wait()

        def extract(jj, slot, jcol, ob):
            gb = gbuf.at[slot]
            col = jnp.full((L,), read_idx(jj) & (BLK - 1), jnp.int32)
            a = plsc.load_gather(gb, [liota, col])
            b = plsc.load_gather(gb, [liota + L, col])
            sv = jnp.sum(a * a + b * b)
            y = _rsqrt_newton(jnp.maximum(
                jnp.full((L,), sv, jnp.float32), 1e-24))
            jv = jnp.full((L,), jcol, jnp.int32)
            plsc.store_scatter(ob, [liota, jv], a * y)
            plsc.store_scatter(ob, [liota + L, jv], b * y)

        fetch(0, 0)
        fetch(1, 1)
        out_copies = []
        for c in range(n_chunks):
            ob = obuf.at[c % 2]
            if c >= 2:
                out_copies[c - 2].wait()

            def body(i, _):
                jj = c * CHUNK + 2 * i
                drain(0)
                fetch(jj + 2, 0)
                extract(jj, 0, 2 * i, ob)
                drain(1)
                fetch(jj + 3, 1)
                extract(jj + 1, 1, 2 * i + 1, ob)
                return 0

            lax.fori_loop(0, CHUNK // 2, body, 0)
            out_copies.append(pltpu.async_copy(
                ob, out_hbm.at[:, pl.ds(base + c * CHUNK, CHUNK)], osem))
        drain(0)
        drain(1)
        out_copies[-2].wait()
        out_copies[-1].wait()

    return sc_kernel(task_ids.astype(jnp.int32), table.T).T


# speedup vs baseline: 3.9250x; 1.7551x over previous
import functools

import jax
import jax.numpy as jnp
from jax import lax
from jax.experimental import pallas as pl
from jax.experimental.pallas import tpu as pltpu
from jax.experimental.pallas import tpu_sc as plsc


def _rsqrt_newton(sv):
    ih = lax.bitcast_convert_type(sv, jnp.int32)
    ih = jnp.int32(0x5F3759DF) - lax.shift_right_logical(ih, 1)
    y = lax.bitcast_convert_type(ih, jnp.float32)
    for _ in range(3):
        y = y * (1.5 - 0.5 * sv * y * y)
    return y


def kernel(task_ids, table):
    B, = task_ids.shape
    V, D = table.shape
    info = plsc.get_sparse_core_info()
    NC, NS, L = info.num_cores, info.num_subcores, info.num_lanes
    NW = NC * NS
    b_per_w = B // NW
    CHUNK = 128
    n_chunks = b_per_w // CHUNK
    BLK = 128
    NBUF = 16

    mesh = plsc.VectorSubcoreMesh(core_axis_name="c", subcore_axis_name="s")

    @functools.partial(
        pl.kernel,
        out_type=jax.ShapeDtypeStruct((D, B), jnp.float32),
        mesh=mesh,
        compiler_params=pltpu.CompilerParams(needs_layout_passes=False),
        scratch_types=[
            pltpu.VMEM((b_per_w,), jnp.int32),
            pltpu.VMEM((NBUF, D, BLK), jnp.float32),
            pltpu.VMEM((2, D, CHUNK), jnp.float32),
        ] + [pltpu.SemaphoreType.DMA] * NBUF + [
            pltpu.SemaphoreType.DMA,
        ],
    )
    def sc_kernel(idx_hbm, tab_hbm, out_hbm, idx_v, gbuf, obuf, *sems):
        wid = lax.axis_index("s") * NC + lax.axis_index("c")
        base = wid * b_per_w
        gsems = sems[:NBUF]
        osem = sems[NBUF]

        pltpu.sync_copy(idx_hbm.at[pl.ds(base, b_per_w)], idx_v)
        liota = lax.iota(jnp.int32, L)

        def read_idx(jj):
            grp = idx_v[pl.ds((jj // L) * L, L)]
            return jnp.sum(jnp.where(liota == jj % L, grp, 0))

        def fetch(jj, slot):
            v = read_idx(jnp.minimum(jj, b_per_w - 1))
            vb = pl.multiple_of((v >> 7) * BLK, BLK)
            pltpu.async_copy(tab_hbm.at[:, pl.ds(vb, BLK)],
                             gbuf.at[slot], gsems[slot])

        def drain(slot):
            pltpu.make_async_copy(tab_hbm.at[:, pl.ds(0, BLK)],
                                  gbuf.at[slot], gsems[slot]).wait()

        def extract(jj, slot, jcol, ob):
            gb = gbuf.at[slot]
            col = jnp.full((L,), read_idx(jj) & (BLK - 1), jnp.int32)
            a = plsc.load_gather(gb, [liota, col])
            b = plsc.load_gather(gb, [liota + L, col])
            sv = jnp.sum(a * a + b * b)
            y = _rsqrt_newton(jnp.maximum(
                jnp.full((L,), sv, jnp.float32), 1e-24))
            jv = jnp.full((L,), jcol, jnp.int32)
            plsc.store_scatter(ob, [liota, jv], a * y)
            plsc.store_scatter(ob, [liota + L, jv], b * y)

        for k in range(NBUF):
            fetch(k, k)
        out_copies = []
        for c in range(n_chunks):
            ob = obuf.at[c % 2]
            if c >= 2:
                out_copies[c - 2].wait()

            def body(i, _):
                jj = c * CHUNK + NBUF * i
                for k in range(NBUF):
                    drain(k)
                    fetch(jj + k + NBUF, k)
                    extract(jj + k, k, NBUF * i + k, ob)
                return 0

            lax.fori_loop(0, CHUNK // NBUF, body, 0)
            out_copies.append(pltpu.async_copy(
                ob, out_hbm.at[:, pl.ds(base + c * CHUNK, CHUNK)], osem))
        for k in range(NBUF):
            drain(k)
        out_copies[-2].wait()
        out_copies[-1].wait()

    return sc_kernel(task_ids.astype(jnp.int32), table.T).T
